# baseline (speedup 1.0000x reference)
import jax
import jax.numpy as jnp
from jax import lax
from jax.experimental import pallas as pl
from jax.experimental.pallas import tpu as pltpu, tpu_sc as plsc

D_MODEL = 768
D_FF = 1024
N_EXP = 16
T = 2048
TM = 192
NC = 2
NS = 16
NW = NC * NS
BPW = T // NW
CH = 16
NCHUNK = BPW // CH


RB = 256
NBLK = T // RB


def _route_body(x_ref, wg_ref, bg_ref, pos_ref, offs_ref, oh_ref, pref_ref):
    x = x_ref[...]
    logits = lax.dot_general(x, wg_ref[...], (((1,), (1,)), ((), ())),
                             preferred_element_type=jnp.float32)
    logits = logits + bg_ref[...]
    iota_e = lax.broadcasted_iota(jnp.int32, (T, N_EXP), 1).astype(jnp.float32)
    m = jnp.max(logits, axis=1, keepdims=True)
    eidf = jnp.min(jnp.where(logits == m, iota_e, float(N_EXP)), axis=1,
                   keepdims=True)
    onehot = (iota_e == eidf).astype(jnp.float32)
    oh_ref[...] = onehot
    bc = jnp.sum(onehot.reshape(NBLK, RB, N_EXP), axis=1)
    tri_b = (lax.broadcasted_iota(jnp.int32, (NBLK, NBLK), 1)
             < lax.broadcasted_iota(jnp.int32, (NBLK, NBLK), 0))
    pref = lax.dot_general(tri_b.astype(jnp.float32), bc,
                           (((1,), (0,)), ((), ())),
                           preferred_element_type=jnp.float32)
    pref_ref[...] = pref
    counts = jnp.sum(bc, axis=0, keepdims=True)
    tri_e = (lax.broadcasted_iota(jnp.int32, (N_EXP, N_EXP), 0)
             < lax.broadcasted_iota(jnp.int32, (N_EXP, N_EXP), 1))
    offs = lax.dot_general(counts, tri_e.astype(jnp.float32),
                           (((1,), (0,)), ((), ())),
                           preferred_element_type=jnp.float32)
    offs_ref[...] = offs.astype(jnp.int32)

    a = (lax.broadcasted_iota(jnp.int32, (RB, RB), 1)
         < lax.broadcasted_iota(jnp.int32, (RB, RB), 0)).astype(jnp.float32)

    def blk(b, _):
        oh = oh_ref[pl.ds(b * RB, RB), :]
        rank = lax.dot_general(a, oh, (((1,), (0,)), ((), ())),
                               preferred_element_type=jnp.float32)
        rank = rank + pref_ref[pl.ds(b, 1), :]
        posf = jnp.sum(oh * (offs + rank), axis=1)
        pos_ref[pl.ds(b, 1), :] = posf.astype(jnp.int32).reshape(1, RB)
        return 0

    lax.fori_loop(0, NBLK, blk, 0)


def _route(x2d, wg, bg2):
    return pl.pallas_call(
        _route_body,
        out_shape=(
            jax.ShapeDtypeStruct((NBLK, RB), jnp.int32),
            jax.ShapeDtypeStruct((1, N_EXP), jnp.int32),
        ),
        scratch_shapes=[pltpu.VMEM((T, N_EXP), jnp.float32),
                        pltpu.VMEM((NBLK, N_EXP), jnp.float32)],
    )(x2d, wg, bg2)


def _mlp_body(offs_ref, xs_ref, w1_ref, b1_ref, w2_ref, b2_ref, out_ref):
    e = pl.program_id(0)
    start = offs_ref[0, e]
    end = jnp.where(e == N_EXP - 1, T,
                    offs_ref[0, jnp.minimum(e + 1, N_EXP - 1)])
    base0 = (start // 8) * 8
    ntiles = (end - base0 + TM - 1) // TM

    def tile(i, _):
        base = jnp.minimum(base0 + i * TM, T - TM)
        xt = xs_ref[pl.ds(base, TM), :].astype(jnp.bfloat16)
        w1 = w1_ref[0].astype(jnp.bfloat16)
        h = lax.dot_general(xt, w1, (((1,), (1,)), ((), ())),
                            preferred_element_type=jnp.float32)
        h = jnp.maximum(h + b1_ref[0], 0.0).astype(jnp.bfloat16)
        w2 = w2_ref[0].astype(jnp.bfloat16)
        eo = lax.dot_general(h, w2, (((1,), (1,)), ((), ())),
                             preferred_element_type=jnp.float32)
        eo = eo + b2_ref[0]
        rowid = base + lax.broadcasted_iota(jnp.int32, (TM, 1), 0)
        old = out_ref[pl.ds(base, TM), :]
        out_ref[pl.ds(base, TM), :] = jnp.where(rowid >= start, eo, old)
        return 0

    lax.fori_loop(0, ntiles, tile, 0)


def _mlp(offs, xs, w1, b1r, w2, b2r):
    grid_spec = pltpu.PrefetchScalarGridSpec(
        num_scalar_prefetch=1,
        grid=(N_EXP,),
        in_specs=[
            pl.BlockSpec((T, D_MODEL), lambda e, offs: (0, 0)),
            pl.BlockSpec((1, D_FF, D_MODEL), lambda e, offs: (e, 0, 0)),
            pl.BlockSpec((1, 1, D_FF), lambda e, offs: (e, 0, 0)),
            pl.BlockSpec((1, D_MODEL, D_FF), lambda e, offs: (e, 0, 0)),
            pl.BlockSpec((1, 1, D_MODEL), lambda e, offs: (e, 0, 0)),
        ],
        out_specs=pl.BlockSpec((T, D_MODEL), lambda e, offs: (0, 0)),
    )
    return pl.pallas_call(
        _mlp_body,
        grid_spec=grid_spec,
        out_shape=jax.ShapeDtypeStruct((T, D_MODEL), jnp.float32),
        compiler_params=pltpu.CompilerParams(
            dimension_semantics=("arbitrary",)),
    )(offs, xs, w1, b1r, w2, b2r)


def _sc_scatter(x2d, pos):
    mesh = plsc.VectorSubcoreMesh(core_axis_name="c", subcore_axis_name="s")

    @pl.kernel(
        out_type=jax.ShapeDtypeStruct((T, D_MODEL), jnp.float32),
        mesh=mesh,
        scratch_types=[
            pltpu.VMEM((BPW,), jnp.int32),
            pltpu.VMEM((BPW, D_MODEL), jnp.float32),
            pltpu.SemaphoreType.DMA,
            pltpu.SemaphoreType.DMA,
        ],
    )
    def k(x_hbm, pos_hbm, out_hbm, idx_v, rows_v, sem_ld, sem_st):
        wid = lax.axis_index("s") * NC + lax.axis_index("c")
        base = wid * BPW
        pltpu.sync_copy(pos_hbm.at[pl.ds(base, BPW)], idx_v)
        loads = [
            pltpu.async_copy(x_hbm.at[pl.ds(base + j * CH, CH)],
                             rows_v.at[pl.ds(j * CH, CH)], sem_ld)
            for j in range(NCHUNK)
        ]
        stores = []
        for j in range(NCHUNK):
            loads[j].wait()
            iv = idx_v[pl.ds(j * CH, CH)]
            stores.append(
                pltpu.async_copy(rows_v.at[pl.ds(j * CH, CH)],
                                 out_hbm.at[iv], sem_st))
        for s in stores:
            s.wait()

    return k(x2d, pos)


def _sc_gather(y, pos):
    mesh = plsc.VectorSubcoreMesh(core_axis_name="c", subcore_axis_name="s")

    @pl.kernel(
        out_type=jax.ShapeDtypeStruct((T, D_MODEL), jnp.float32),
        mesh=mesh,
        scratch_types=[
            pltpu.VMEM((BPW,), jnp.int32),
            pltpu.VMEM((BPW, D_MODEL), jnp.float32),
            pltpu.SemaphoreType.DMA,
            pltpu.SemaphoreType.DMA,
        ],
    )
    def k(y_hbm, pos_hbm, out_hbm, idx_v, rows_v, sem_ld, sem_st):
        wid = lax.axis_index("s") * NC + lax.axis_index("c")
        base = wid * BPW
        pltpu.sync_copy(pos_hbm.at[pl.ds(base, BPW)], idx_v)
        gathers = []
        for j in range(NCHUNK):
            iv = idx_v[pl.ds(j * CH, CH)]
            gathers.append(
                pltpu.async_copy(y_hbm.at[iv],
                                 rows_v.at[pl.ds(j * CH, CH)], sem_ld))
        stores = []
        for j in range(NCHUNK):
            gathers[j].wait()
            stores.append(
                pltpu.async_copy(rows_v.at[pl.ds(j * CH, CH)],
                                 out_hbm.at[pl.ds(base + j * CH, CH)], sem_st))
        for s in stores:
            s.wait()

    return k(y, pos)


def kernel(x, Wg, bg, W1, b1, W2, b2):
    B, S, D = x.shape
    x2d = x.reshape(T, D_MODEL)
    pos2d, offs = _route(x2d, Wg, bg.reshape(1, N_EXP))
    pos = pos2d.reshape(T)
    xs = _sc_scatter(x2d, pos)
    y = _mlp(offs, xs, W1, b1.reshape(N_EXP, 1, D_FF),
             W2, b2.reshape(N_EXP, 1, D_MODEL))
    out = _sc_gather(y, pos)
    return out.reshape(B, S, D)


# speedup vs baseline: 1.0183x; 1.0183x over previous
import jax
import jax.numpy as jnp
from jax import lax
from jax.experimental import pallas as pl
from jax.experimental.pallas import tpu as pltpu, tpu_sc as plsc

D_MODEL = 768
D_FF = 1024
N_EXP = 16
T = 2048
TM = 192
NC = 2
NS = 16
NW = NC * NS
BPW = T // NW
CH = 16
NCHUNK = BPW // CH


RB = 128
NBLK = T // RB


def _route_body(x_ref, wg_ref, bg_ref, pos_ref, offs_ref, oh_ref, pref_ref):
    x = x_ref[...]
    logits = lax.dot_general(x, wg_ref[...], (((1,), (1,)), ((), ())),
                             preferred_element_type=jnp.float32)
    logits = logits + bg_ref[...]
    iota_e = lax.broadcasted_iota(jnp.int32, (T, N_EXP), 1).astype(jnp.float32)
    m = jnp.max(logits, axis=1, keepdims=True)
    eidf = jnp.min(jnp.where(logits == m, iota_e, float(N_EXP)), axis=1,
                   keepdims=True)
    onehot = (iota_e == eidf).astype(jnp.float32)
    oh_ref[...] = onehot
    bc = jnp.sum(onehot.reshape(NBLK, RB, N_EXP), axis=1)
    tri_b = (lax.broadcasted_iota(jnp.int32, (NBLK, NBLK), 1)
             < lax.broadcasted_iota(jnp.int32, (NBLK, NBLK), 0))
    pref = lax.dot_general(tri_b.astype(jnp.float32), bc,
                           (((1,), (0,)), ((), ())),
                           preferred_element_type=jnp.float32)
    pref_ref[...] = pref
    counts = jnp.sum(bc, axis=0, keepdims=True)
    tri_e = (lax.broadcasted_iota(jnp.int32, (N_EXP, N_EXP), 0)
             < lax.broadcasted_iota(jnp.int32, (N_EXP, N_EXP), 1))
    offs = lax.dot_general(counts, tri_e.astype(jnp.float32),
                           (((1,), (0,)), ((), ())),
                           preferred_element_type=jnp.float32)
    offs_ref[...] = offs.astype(jnp.int32)

    a = (lax.broadcasted_iota(jnp.int32, (RB, RB), 1)
         < lax.broadcasted_iota(jnp.int32, (RB, RB), 0)).astype(jnp.float32)

    def blk(b, _):
        oh = oh_ref[pl.ds(b * RB, RB), :]
        rank = lax.dot_general(a, oh, (((1,), (0,)), ((), ())),
                               preferred_element_type=jnp.float32)
        rank = rank + pref_ref[pl.ds(b, 1), :]
        posf = jnp.sum(oh * (offs + rank), axis=1)
        pos_ref[pl.ds(b, 1), :] = posf.astype(jnp.int32).reshape(1, RB)
        return 0

    lax.fori_loop(0, NBLK, blk, 0)


def _route(x2d, wg, bg2):
    return pl.pallas_call(
        _route_body,
        out_shape=(
            jax.ShapeDtypeStruct((NBLK, RB), jnp.int32),
            jax.ShapeDtypeStruct((1, N_EXP), jnp.int32),
        ),
        scratch_shapes=[pltpu.VMEM((T, N_EXP), jnp.float32),
                        pltpu.VMEM((NBLK, N_EXP), jnp.float32)],
    )(x2d, wg, bg2)


def _mlp_body(offs_ref, xs_ref, w1_ref, b1_ref, w2_ref, b2_ref, out_ref):
    e = pl.program_id(0)
    start = offs_ref[0, e]
    end = jnp.where(e == N_EXP - 1, T,
                    offs_ref[0, jnp.minimum(e + 1, N_EXP - 1)])
    base0 = (start // 8) * 8
    ntiles = (end - base0 + TM - 1) // TM

    def tile(i, _):
        base = jnp.minimum(base0 + i * TM, T - TM)
        xt = xs_ref[pl.ds(base, TM), :].astype(jnp.bfloat16)
        w1 = w1_ref[0].astype(jnp.bfloat16)
        h = lax.dot_general(xt, w1, (((1,), (1,)), ((), ())),
                            preferred_element_type=jnp.float32)
        h = jnp.maximum(h + b1_ref[0], 0.0).astype(jnp.bfloat16)
        w2 = w2_ref[0].astype(jnp.bfloat16)
        eo = lax.dot_general(h, w2, (((1,), (1,)), ((), ())),
                             preferred_element_type=jnp.float32)
        eo = eo + b2_ref[0]
        rowid = base + lax.broadcasted_iota(jnp.int32, (TM, 1), 0)
        old = out_ref[pl.ds(base, TM), :]
        out_ref[pl.ds(base, TM), :] = jnp.where(rowid >= start, eo, old)
        return 0

    lax.fori_loop(0, ntiles, tile, 0)


def _mlp(offs, xs, w1, b1r, w2, b2r):
    grid_spec = pltpu.PrefetchScalarGridSpec(
        num_scalar_prefetch=1,
        grid=(N_EXP,),
        in_specs=[
            pl.BlockSpec((T, D_MODEL), lambda e, offs: (0, 0)),
            pl.BlockSpec((1, D_FF, D_MODEL), lambda e, offs: (e, 0, 0)),
            pl.BlockSpec((1, 1, D_FF), lambda e, offs: (e, 0, 0)),
            pl.BlockSpec((1, D_MODEL, D_FF), lambda e, offs: (e, 0, 0)),
            pl.BlockSpec((1, 1, D_MODEL), lambda e, offs: (e, 0, 0)),
        ],
        out_specs=pl.BlockSpec((T, D_MODEL), lambda e, offs: (0, 0)),
    )
    return pl.pallas_call(
        _mlp_body,
        grid_spec=grid_spec,
        out_shape=jax.ShapeDtypeStruct((T, D_MODEL), jnp.float32),
        compiler_params=pltpu.CompilerParams(
            dimension_semantics=("arbitrary",)),
    )(offs, xs, w1, b1r, w2, b2r)


def _sc_scatter(x2d, pos):
    mesh = plsc.VectorSubcoreMesh(core_axis_name="c", subcore_axis_name="s")

    @pl.kernel(
        out_type=jax.ShapeDtypeStruct((T, D_MODEL), jnp.float32),
        mesh=mesh,
        scratch_types=[
            pltpu.VMEM((BPW,), jnp.int32),
            pltpu.VMEM((BPW, D_MODEL), jnp.float32),
            pltpu.SemaphoreType.DMA,
            pltpu.SemaphoreType.DMA,
        ],
    )
    def k(x_hbm, pos_hbm, out_hbm, idx_v, rows_v, sem_ld, sem_st):
        wid = lax.axis_index("s") * NC + lax.axis_index("c")
        base = wid * BPW
        pltpu.sync_copy(pos_hbm.at[pl.ds(base, BPW)], idx_v)
        loads = [
            pltpu.async_copy(x_hbm.at[pl.ds(base + j * CH, CH)],
                             rows_v.at[pl.ds(j * CH, CH)], sem_ld)
            for j in range(NCHUNK)
        ]
        stores = []
        for j in range(NCHUNK):
            loads[j].wait()
            iv = idx_v[pl.ds(j * CH, CH)]
            stores.append(
                pltpu.async_copy(rows_v.at[pl.ds(j * CH, CH)],
                                 out_hbm.at[iv], sem_st))
        for s in stores:
            s.wait()

    return k(x2d, pos)


def _sc_gather(y, pos):
    mesh = plsc.VectorSubcoreMesh(core_axis_name="c", subcore_axis_name="s")

    @pl.kernel(
        out_type=jax.ShapeDtypeStruct((T, D_MODEL), jnp.float32),
        mesh=mesh,
        scratch_types=[
            pltpu.VMEM((BPW,), jnp.int32),
            pltpu.VMEM((BPW, D_MODEL), jnp.float32),
            pltpu.SemaphoreType.DMA,
            pltpu.SemaphoreType.DMA,
        ],
    )
    def k(y_hbm, pos_hbm, out_hbm, idx_v, rows_v, sem_ld, sem_st):
        wid = lax.axis_index("s") * NC + lax.axis_index("c")
        base = wid * BPW
        pltpu.sync_copy(pos_hbm.at[pl.ds(base, BPW)], idx_v)
        gathers = []
        for j in range(NCHUNK):
            iv = idx_v[pl.ds(j * CH, CH)]
            gathers.append(
                pltpu.async_copy(y_hbm.at[iv],
                                 rows_v.at[pl.ds(j * CH, CH)], sem_ld))
        stores = []
        for j in range(NCHUNK):
            gathers[j].wait()
            stores.append(
                pltpu.async_copy(rows_v.at[pl.ds(j * CH, CH)],
                                 out_hbm.at[pl.ds(base + j * CH, CH)], sem_st))
        for s in stores:
            s.wait()

    return k(y, pos)


def kernel(x, Wg, bg, W1, b1, W2, b2):
    B, S, D = x.shape
    x2d = x.reshape(T, D_MODEL)
    pos2d, offs = _route(x2d, Wg, bg.reshape(1, N_EXP))
    pos = pos2d.reshape(T)
    xs = _sc_scatter(x2d, pos)
    y = _mlp(offs, xs, W1, b1.reshape(N_EXP, 1, D_FF),
             W2, b2.reshape(N_EXP, 1, D_MODEL))
    out = _sc_gather(y, pos)
    return out.reshape(B, S, D)
